# baseline (speedup 1.0000x reference)
import jax
import jax.numpy as jnp
from jax.experimental import pallas as pl

_BATCH = 128
_VOCAB = 200
_DIM = 128


def _bcast_kernel(w_ref, out_ref):
    out_ref[...] = jnp.broadcast_to(w_ref[...][None, :, :],
                                    (_BATCH, _VOCAB, _DIM))


def kernel(x, pe_weight):
    del x
    return pl.pallas_call(
        _bcast_kernel,
        out_shape=jax.ShapeDtypeStruct((_BATCH, _VOCAB, _DIM), jnp.float32),
    )(pe_weight)
